# baseline (speedup 1.0000x reference)
import jax
import jax.numpy as jnp
from jax.experimental import pallas as pl
from jax.experimental.pallas import tpu as pltpu

_NCLS = 40
_BLK = 1024


def _fused_kernel(starts_ref, ends_ref, invc_ref, x_ref,
                  W1_ref, b1_ref, W2_ref, b2_ref, W3_ref, b3_ref,
                  sums_ref, logits_ref):
    i = pl.program_id(0)
    nb = pl.num_programs(0)
    B = starts_ref.shape[1]

    x = x_ref[...].astype(jnp.bfloat16)
    h = jnp.maximum(
        jnp.dot(x, W1_ref[...], preferred_element_type=jnp.float32)
        + b1_ref[...], 0.0).astype(jnp.bfloat16)
    o = jnp.maximum(
        jnp.dot(h, W2_ref[...], preferred_element_type=jnp.float32)
        + b2_ref[...], 0.0).astype(jnp.bfloat16)
    logits = (jnp.dot(o, W3_ref[...], preferred_element_type=jnp.float32)
              + b3_ref[...])
    logits_ref[...] = logits

    row = i * _BLK + jax.lax.broadcasted_iota(jnp.int32, (_BLK, B), 0)
    onehot = ((row >= starts_ref[...]) & (row < ends_ref[...])
              ).astype(jnp.float32)
    part = jax.lax.dot_general(
        onehot, logits, (((0,), (0,)), ((), ())),
        preferred_element_type=jnp.float32)

    @pl.when(i == 0)
    def _():
        sums_ref[...] = jnp.zeros_like(sums_ref)

    sums_ref[...] += part

    @pl.when(i == nb - 1)
    def _():
        sums_ref[...] = sums_ref[...] * invc_ref[...]


def kernel(x, cu_seqlens, W1, b1, W2, b2, W3, b3):
    N, D = x.shape
    H = W1.shape[1]
    E = W2.shape[1]
    B = cu_seqlens.shape[0] - 1

    starts = cu_seqlens[:-1].reshape(1, B)
    ends = cu_seqlens[1:].reshape(1, B)
    inv_counts = (1.0 / jnp.maximum(
        (ends - starts).astype(jnp.float32), 1.0)).reshape(B, 1)

    nb = N // _BLK
    grid = (nb,)

    full = lambda shape: pl.BlockSpec(shape, lambda i: (0, 0))

    global_logits, logits = pl.pallas_call(
        _fused_kernel,
        grid=grid,
        in_specs=[
            full((1, B)),
            full((1, B)),
            full((B, 1)),
            pl.BlockSpec((_BLK, D), lambda i: (i, 0)),
            full((D, H)),
            full((1, H)),
            full((H, E)),
            full((1, E)),
            full((E, _NCLS)),
            full((1, _NCLS)),
        ],
        out_specs=[
            full((B, _NCLS)),
            pl.BlockSpec((_BLK, _NCLS), lambda i: (i, 0)),
        ],
        out_shape=[
            jax.ShapeDtypeStruct((B, _NCLS), jnp.float32),
            jax.ShapeDtypeStruct((N, _NCLS), jnp.float32),
        ],
        compiler_params=pltpu.CompilerParams(
            dimension_semantics=("arbitrary",)),
    )(starts, ends, inv_counts, x,
      W1.astype(jnp.bfloat16), b1.reshape(1, H),
      W2.astype(jnp.bfloat16), b2.reshape(1, E),
      W3.astype(jnp.bfloat16), b3.reshape(1, _NCLS))

    return (global_logits, logits)


# speedup vs baseline: 1.0948x; 1.0948x over previous
import jax
import jax.numpy as jnp
from jax.experimental import pallas as pl
from jax.experimental.pallas import tpu as pltpu

_NCLS = 40
_BLK = 1024


def _fused_kernel(starts_ref, ends_ref, invc_ref, x_ref,
                  W1_ref, b1_ref, W2_ref, b2_ref, W3_ref, b3_ref,
                  sums_ref, logits_ref):
    i = pl.program_id(0)
    nb = pl.num_programs(0)
    B = starts_ref.shape[1]

    x = x_ref[...]
    h = jnp.maximum(
        jnp.dot(x, W1_ref[...], preferred_element_type=jnp.float32)
        + b1_ref[...], 0.0)
    o = jnp.maximum(
        jnp.dot(h, W2_ref[...], preferred_element_type=jnp.float32)
        + b2_ref[...], 0.0)
    logits = (jnp.dot(o, W3_ref[...], preferred_element_type=jnp.float32)
              + b3_ref[...])
    logits_ref[...] = logits

    row = i * _BLK + jax.lax.broadcasted_iota(jnp.int32, (_BLK, B), 0)
    onehot = ((row >= starts_ref[...]) & (row < ends_ref[...])
              ).astype(jnp.float32)
    part = jax.lax.dot_general(
        onehot, logits, (((0,), (0,)), ((), ())),
        preferred_element_type=jnp.float32)

    @pl.when(i == 0)
    def _():
        sums_ref[...] = jnp.zeros_like(sums_ref)

    sums_ref[...] += part

    @pl.when(i == nb - 1)
    def _():
        sums_ref[...] = sums_ref[...] * invc_ref[...]


def kernel(x, cu_seqlens, W1, b1, W2, b2, W3, b3):
    N, D = x.shape
    H = W1.shape[1]
    E = W2.shape[1]
    B = cu_seqlens.shape[0] - 1

    starts = cu_seqlens[:-1].reshape(1, B)
    ends = cu_seqlens[1:].reshape(1, B)
    inv_counts = (1.0 / jnp.maximum(
        (ends - starts).astype(jnp.float32), 1.0)).reshape(B, 1)

    nb = N // _BLK
    grid = (nb,)

    full = lambda shape: pl.BlockSpec(shape, lambda i: (0, 0))

    global_logits, logits = pl.pallas_call(
        _fused_kernel,
        grid=grid,
        in_specs=[
            full((1, B)),
            full((1, B)),
            full((B, 1)),
            pl.BlockSpec((_BLK, D), lambda i: (i, 0)),
            full((D, H)),
            full((1, H)),
            full((H, E)),
            full((1, E)),
            full((E, _NCLS)),
            full((1, _NCLS)),
        ],
        out_specs=[
            full((B, _NCLS)),
            pl.BlockSpec((_BLK, _NCLS), lambda i: (i, 0)),
        ],
        out_shape=[
            jax.ShapeDtypeStruct((B, _NCLS), jnp.float32),
            jax.ShapeDtypeStruct((N, _NCLS), jnp.float32),
        ],
        compiler_params=pltpu.CompilerParams(
            dimension_semantics=("arbitrary",)),
    )(starts, ends, inv_counts, x,
      W1, b1.reshape(1, H),
      W2, b2.reshape(1, E),
      W3, b3.reshape(1, _NCLS))

    return (global_logits, logits)


# speedup vs baseline: 1.2666x; 1.1569x over previous
import jax
import jax.numpy as jnp
from jax.experimental import pallas as pl
from jax.experimental.pallas import tpu as pltpu

_NCLS = 40
_BLK = 2048


def _fused_kernel(starts_ref, ends_ref, invc_ref, x_ref,
                  W1_ref, b1_ref, W2_ref, b2_ref, W3_ref, b3_ref,
                  sums_ref, logits_ref):
    i = pl.program_id(0)
    nb = pl.num_programs(0)
    B = starts_ref.shape[1]

    x = x_ref[...]
    h = jnp.maximum(
        jnp.dot(x, W1_ref[...], preferred_element_type=jnp.float32)
        + b1_ref[...], 0.0)
    o = jnp.maximum(
        jnp.dot(h, W2_ref[...], preferred_element_type=jnp.float32)
        + b2_ref[...], 0.0)
    logits = (jnp.dot(o, W3_ref[...], preferred_element_type=jnp.float32)
              + b3_ref[...])
    logits_ref[...] = logits

    row = i * _BLK + jax.lax.broadcasted_iota(jnp.int32, (_BLK, B), 0)
    onehot = ((row >= starts_ref[...]) & (row < ends_ref[...])
              ).astype(jnp.float32)
    part = jax.lax.dot_general(
        onehot, logits, (((0,), (0,)), ((), ())),
        preferred_element_type=jnp.float32)

    @pl.when(i == 0)
    def _():
        sums_ref[...] = jnp.zeros_like(sums_ref)

    sums_ref[...] += part

    @pl.when(i == nb - 1)
    def _():
        sums_ref[...] = sums_ref[...] * invc_ref[...]


def kernel(x, cu_seqlens, W1, b1, W2, b2, W3, b3):
    N, D = x.shape
    H = W1.shape[1]
    E = W2.shape[1]
    B = cu_seqlens.shape[0] - 1

    starts = cu_seqlens[:-1].reshape(1, B)
    ends = cu_seqlens[1:].reshape(1, B)
    inv_counts = (1.0 / jnp.maximum(
        (ends - starts).astype(jnp.float32), 1.0)).reshape(B, 1)

    nb = N // _BLK
    grid = (nb,)

    full = lambda shape: pl.BlockSpec(shape, lambda i: (0, 0))

    global_logits, logits = pl.pallas_call(
        _fused_kernel,
        grid=grid,
        in_specs=[
            full((1, B)),
            full((1, B)),
            full((B, 1)),
            pl.BlockSpec((_BLK, D), lambda i: (i, 0)),
            full((D, H)),
            full((1, H)),
            full((H, E)),
            full((1, E)),
            full((E, _NCLS)),
            full((1, _NCLS)),
        ],
        out_specs=[
            full((B, _NCLS)),
            pl.BlockSpec((_BLK, _NCLS), lambda i: (i, 0)),
        ],
        out_shape=[
            jax.ShapeDtypeStruct((B, _NCLS), jnp.float32),
            jax.ShapeDtypeStruct((N, _NCLS), jnp.float32),
        ],
        compiler_params=pltpu.CompilerParams(
            dimension_semantics=("arbitrary",)),
    )(starts, ends, inv_counts, x,
      W1, b1.reshape(1, H),
      W2, b2.reshape(1, E),
      W3, b3.reshape(1, _NCLS))

    return (global_logits, logits)


# speedup vs baseline: 1.2910x; 1.0192x over previous
import jax
import jax.numpy as jnp
from jax.experimental import pallas as pl
from jax.experimental.pallas import tpu as pltpu

_NCLS = 40
_BLK = 4096


def _fused_kernel(starts_ref, ends_ref, invc_ref, x_ref,
                  W1_ref, b1_ref, W2_ref, b2_ref, W3_ref, b3_ref,
                  sums_ref, logits_ref):
    i = pl.program_id(0)
    nb = pl.num_programs(0)
    B = starts_ref.shape[1]

    x = x_ref[...]
    h = jnp.maximum(
        jnp.dot(x, W1_ref[...], preferred_element_type=jnp.float32)
        + b1_ref[...], 0.0)
    o = jnp.maximum(
        jnp.dot(h, W2_ref[...], preferred_element_type=jnp.float32)
        + b2_ref[...], 0.0)
    logits = (jnp.dot(o, W3_ref[...], preferred_element_type=jnp.float32)
              + b3_ref[...])
    logits_ref[...] = logits

    row = i * _BLK + jax.lax.broadcasted_iota(jnp.int32, (_BLK, B), 0)
    onehot = ((row >= starts_ref[...]) & (row < ends_ref[...])
              ).astype(jnp.float32)
    part = jax.lax.dot_general(
        onehot, logits, (((0,), (0,)), ((), ())),
        preferred_element_type=jnp.float32)

    @pl.when(i == 0)
    def _():
        sums_ref[...] = jnp.zeros_like(sums_ref)

    sums_ref[...] += part

    @pl.when(i == nb - 1)
    def _():
        sums_ref[...] = sums_ref[...] * invc_ref[...]


def kernel(x, cu_seqlens, W1, b1, W2, b2, W3, b3):
    N, D = x.shape
    H = W1.shape[1]
    E = W2.shape[1]
    B = cu_seqlens.shape[0] - 1

    starts = cu_seqlens[:-1].reshape(1, B)
    ends = cu_seqlens[1:].reshape(1, B)
    inv_counts = (1.0 / jnp.maximum(
        (ends - starts).astype(jnp.float32), 1.0)).reshape(B, 1)

    nb = N // _BLK
    grid = (nb,)

    full = lambda shape: pl.BlockSpec(shape, lambda i: (0, 0))

    global_logits, logits = pl.pallas_call(
        _fused_kernel,
        grid=grid,
        in_specs=[
            full((1, B)),
            full((1, B)),
            full((B, 1)),
            pl.BlockSpec((_BLK, D), lambda i: (i, 0)),
            full((D, H)),
            full((1, H)),
            full((H, E)),
            full((1, E)),
            full((E, _NCLS)),
            full((1, _NCLS)),
        ],
        out_specs=[
            full((B, _NCLS)),
            pl.BlockSpec((_BLK, _NCLS), lambda i: (i, 0)),
        ],
        out_shape=[
            jax.ShapeDtypeStruct((B, _NCLS), jnp.float32),
            jax.ShapeDtypeStruct((N, _NCLS), jnp.float32),
        ],
        compiler_params=pltpu.CompilerParams(
            dimension_semantics=("arbitrary",)),
    )(starts, ends, inv_counts, x,
      W1, b1.reshape(1, H),
      W2, b2.reshape(1, E),
      W3, b3.reshape(1, _NCLS))

    return (global_logits, logits)
